# baseline (speedup 1.0000x reference)
import jax
import jax.numpy as jnp
from jax.experimental import pallas as pl

N = 3960
U = 64
B = 4
R = 360
NB = N // R
BU = B * U

f32 = jnp.float32
bf16 = jnp.bfloat16
_HI = jax.lax.Precision.HIGHEST


def _dot(a, b, precision=_HI):
    return jax.lax.dot_general(a, b, (((1,), (0,)), ((), ())),
                               precision=precision,
                               preferred_element_type=f32)


def _dot_t(a, b, precision=_HI):
    return jax.lax.dot_general(a, b, (((0,), (0,)), ((), ())),
                               precision=precision,
                               preferred_element_type=f32)


def _inv(d):
    return jnp.where(d > 0, 1.0 / d, 0.0)


def _p1_body(a_ref, xe_full_ref, xe_blk_ref, p116_ref, sf_ref, sb_ref):
    i = pl.program_id(0)
    a = a_ref[...]
    d_blk = jnp.sum(a, axis=1, keepdims=True)
    p116_ref[...] = (a * _inv(d_blk)).astype(bf16)
    sf_ref[...] = _dot(a, xe_full_ref[...])
    bwd = _dot_t(a, xe_blk_ref[...])

    @pl.when(i == 0)
    def _():
        sb_ref[...] = bwd

    @pl.when(i != 0)
    def _():
        sb_ref[...] += bwd


def _p2_body(a_ref, dcinv_row_ref, sfe_full_ref, sfe_blk_ref, sbe_blk_ref,
             x_blk_ref, q16_ref, t2f_ref, t2b_ref):
    i = pl.program_id(0)
    a = a_ref[...]
    q16_ref[...] = (a * dcinv_row_ref[...]).astype(bf16)
    sfe = sfe_full_ref[...]
    t1f_full = sfe[:, :4] * _inv(sfe[:, 4:5])
    dinv_blk = _inv(sfe_blk_ref[:, 4:5])
    t2f_ref[...] = 2.0 * dinv_blk * _dot(a, t1f_full) - x_blk_ref[...]
    sbe_blk = sbe_blk_ref[...]
    t1b_blk = sbe_blk[:, :4] * _inv(sbe_blk[:, 4:5])
    bwd = _dot_t(a, t1b_blk)

    @pl.when(i == 0)
    def _():
        t2b_ref[...] = bwd

    @pl.when(i != 0)
    def _():
        t2b_ref[...] += bwd


def _pw0_body(x_ref, sfe_ref, t2f_ref, sbe_ref, t2braw_ref,
              w5u_ref, w5c_ref, bgu_ref, bc_ref, h32_ref, h16_ref):
    x = x_ref[...]
    sfe = sfe_ref[...]
    sbe = sbe_ref[...]
    dinv = _inv(sfe[:, 4:5])
    dcinv = _inv(sbe[:, 4:5])
    t1f = sfe[:, :4] * dinv
    t1b = sbe[:, :4] * dcinv
    t2f = t2f_ref[...]
    t2b = 2.0 * dcinv * t2braw_ref[...] - x
    terms = (x, t1f, t2f, t1b, t2b)
    for b in range(B):
        accu = jnp.broadcast_to(bgu_ref[...], (R, U))
        accc = jnp.broadcast_to(bc_ref[...], (R, U))
        for m, t in enumerate(terms):
            col = t[:, b:b + 1].astype(bf16).astype(f32)
            accu = accu + col * w5u_ref[m:m + 1, :].astype(f32)
            accc = accc + col * w5c_ref[m:m + 1, :].astype(f32)
        u = jax.nn.sigmoid(accu)
        c = jnp.tanh(accc)
        h1 = (1.0 - u) * c
        h32_ref[:, b * U:(b + 1) * U] = h1
        h16_ref[:, b * U:(b + 1) * U] = h1.astype(bf16)


def _p3_body(p116_ref, q16_ref, h16_full_ref, h16_blk_ref,
             t1f16_ref, s1b_ref):
    i = pl.program_id(0)
    fwd = _dot(p116_ref[...], h16_full_ref[...], precision=None)
    t1f16_ref[...] = fwd.astype(bf16)
    bwd = _dot_t(q16_ref[...], h16_blk_ref[...], precision=None)

    @pl.when(i == 0)
    def _():
        s1b_ref[...] = bwd

    @pl.when(i != 0)
    def _():
        s1b_ref[...] += bwd


def _p4_body(p116_ref, q16_ref, t1f16_full_ref, h32_blk_ref, s1b_blk_ref,
             t2f16_ref, s2b_ref):
    i = pl.program_id(0)
    fwd = 2.0 * _dot(p116_ref[...], t1f16_full_ref[...], precision=None) \
        - h32_blk_ref[...]
    t2f16_ref[...] = fwd.astype(bf16)
    bwd = _dot_t(q16_ref[...], s1b_blk_ref[...].astype(bf16), precision=None)

    @pl.when(i == 0)
    def _():
        s2b_ref[...] = bwd

    @pl.when(i != 0)
    def _():
        s2b_ref[...] += bwd


def _pw1_body(h32_ref, t1f16_ref, t2f16_ref, s1b_ref, s2b_ref,
              wgu_ref, wc_ref, bgu_ref, bc_ref, wp_ref, bp_ref,
              hid_ref, proj_ref):
    h = h32_ref[...]
    t2b = 2.0 * s2b_ref[...] - h
    h16 = h.astype(bf16)
    t1f16 = t1f16_ref[...]
    t2f16 = t2f16_ref[...]
    t1b16 = s1b_ref[...].astype(bf16)
    t2b16 = t2b.astype(bf16)
    wgu = wgu_ref[...]
    wc = wc_ref[...]
    wp = wp_ref[...].astype(f32)
    for b in range(B):
        sl = slice(b * U, (b + 1) * U)
        xb = jnp.concatenate(
            [h16[:, sl], t1f16[:, sl], t2f16[:, sl], t1b16[:, sl],
             t2b16[:, sl]], axis=1)
        u = jax.nn.sigmoid(_dot(xb, wgu, precision=None) + bgu_ref[...])
        c = jnp.tanh(_dot(xb, wc, precision=None) + bc_ref[...])
        h2b = (1.0 - u) * c
        hid_ref[0, b, :, :] = h[:, sl]
        hid_ref[1, b, :, :] = h2b
        proj_ref[:, b] = (jnp.sum(h2b.astype(bf16).astype(f32) * wp, axis=1)
                          + bp_ref[0, 0])


def _blk(shape):
    return pl.BlockSpec(shape, lambda i: (i,) + (0,) * (len(shape) - 1))


def _full(shape):
    return pl.BlockSpec(shape, lambda i: (0,) * len(shape))


@jax.jit
def _run(x, adj, Wg0, bg0, Wc0, bc0, Wg1, bg1, Wc1, bc1, Wp, bp):
    xe = jnp.concatenate([x, jnp.ones((N, 1), f32)], axis=1)

    idx0 = jnp.arange(5) * (1 + U)
    W5u = Wg0[idx0][:, U:].astype(bf16)
    W5c = Wc0[idx0].astype(bf16)
    idx1 = (jnp.arange(5)[:, None] * (2 * U) + jnp.arange(U)[None, :]).reshape(-1)
    Wg1u = Wg1[idx1][:, U:].astype(bf16)
    Wc1f = Wc1[idx1].astype(bf16)
    bg0u = bg0[U:].reshape(1, U)
    bc0r = bc0.reshape(1, U)
    bg1u = bg1[U:].reshape(1, U)
    bc1r = bc1.reshape(1, U)
    wp_t = Wp.reshape(1, U).astype(bf16)
    bp_r = bp.reshape(1, 1)

    grid = (NB,)

    p116, sf_e, sb_e = pl.pallas_call(
        _p1_body,
        grid=grid,
        in_specs=[_blk((R, N)), _full((N, 5)), _blk((R, 5))],
        out_specs=[_blk((R, N)), _blk((R, 5)), _full((N, 5))],
        out_shape=[jax.ShapeDtypeStruct((N, N), bf16),
                   jax.ShapeDtypeStruct((N, 5), f32),
                   jax.ShapeDtypeStruct((N, 5), f32)],
    )(adj, xe, xe)

    dcinv_row = _inv(sb_e[:, 4]).reshape(1, N)

    q16, t2f0, t2b0raw = pl.pallas_call(
        _p2_body,
        grid=grid,
        in_specs=[_blk((R, N)), _full((1, N)), _full((N, 5)), _blk((R, 5)),
                  _blk((R, 5)), _blk((R, 4))],
        out_specs=[_blk((R, N)), _blk((R, 4)), _full((N, 4))],
        out_shape=[jax.ShapeDtypeStruct((N, N), bf16),
                   jax.ShapeDtypeStruct((N, 4), f32),
                   jax.ShapeDtypeStruct((N, 4), f32)],
    )(adj, dcinv_row, sf_e, sf_e, sb_e, x)

    h32, h16 = pl.pallas_call(
        _pw0_body,
        grid=grid,
        in_specs=[_blk((R, 4)), _blk((R, 5)), _blk((R, 4)), _blk((R, 5)),
                  _blk((R, 4)), _full((5, U)), _full((5, U)),
                  _full((1, U)), _full((1, U))],
        out_specs=[_blk((R, BU)), _blk((R, BU))],
        out_shape=[jax.ShapeDtypeStruct((N, BU), f32),
                   jax.ShapeDtypeStruct((N, BU), bf16)],
    )(x, sf_e, t2f0, sb_e, t2b0raw, W5u, W5c, bg0u, bc0r)

    t1f16, s1b = pl.pallas_call(
        _p3_body,
        grid=grid,
        in_specs=[_blk((R, N)), _blk((R, N)), _full((N, BU)), _blk((R, BU))],
        out_specs=[_blk((R, BU)), _full((N, BU))],
        out_shape=[jax.ShapeDtypeStruct((N, BU), bf16),
                   jax.ShapeDtypeStruct((N, BU), f32)],
    )(p116, q16, h16, h16)

    t2f16, s2b = pl.pallas_call(
        _p4_body,
        grid=grid,
        in_specs=[_blk((R, N)), _blk((R, N)), _full((N, BU)), _blk((R, BU)),
                  _blk((R, BU))],
        out_specs=[_blk((R, BU)), _full((N, BU))],
        out_shape=[jax.ShapeDtypeStruct((N, BU), bf16),
                   jax.ShapeDtypeStruct((N, BU), f32)],
    )(p116, q16, t1f16, h32, s1b)

    hid4, proj = pl.pallas_call(
        _pw1_body,
        grid=grid,
        in_specs=[_blk((R, BU))] * 5 + [
                  _full((5 * U, U)), _full((5 * U, U)),
                  _full((1, U)), _full((1, U)), _full((1, U)),
                  _full((1, 1))],
        out_specs=[pl.BlockSpec((2, B, R, U), lambda i: (0, 0, i, 0)),
                   _blk((R, B))],
        out_shape=[jax.ShapeDtypeStruct((2, B, N, U), f32),
                   jax.ShapeDtypeStruct((N, B), f32)],
    )(h32, t1f16, t2f16, s1b, s2b, Wg1u, Wc1f, bg1u, bc1r, wp_t, bp_r)

    return proj.T, hid4.reshape(2, B, N * U)


def kernel(inputs, adj_mx, forward_index, Wg0, bg0, Wc0, bc0,
           Wg1, bg1, Wc1, bc1, Wp, bp):
    x = inputs.T.astype(f32)
    return _run(x, adj_mx, Wg0, bg0, Wc0, bc0, Wg1, bg1, Wc1, bc1, Wp, bp)


# speedup vs baseline: 1.9569x; 1.9569x over previous
import jax
import jax.numpy as jnp
from jax.experimental import pallas as pl

N = 3960
U = 64
B = 4
R = 360
NB = N // R
BU = B * U

f32 = jnp.float32
bf16 = jnp.bfloat16


def _dot(a, b):
    return jax.lax.dot_general(a, b, (((1,), (0,)), ((), ())),
                               preferred_element_type=f32)


def _dot_t(a, b):
    return jax.lax.dot_general(a, b, (((0,), (0,)), ((), ())),
                               preferred_element_type=f32)


def _inv(d):
    return jnp.where(d > 0, 1.0 / d, 0.0)


def _p1_support(a):
    rs = jnp.sum(a, axis=1, keepdims=True)
    return (a * _inv(rs)).astype(bf16)


def _p1_body(a_ref, x16_full_ref, x_blk_ref, t1f16_ref, cs_ref, b1_ref):
    i = pl.program_id(0)
    a = a_ref[...]
    p116 = _p1_support(a)
    t1f16_ref[...] = _dot(p116, x16_full_ref[...]).astype(bf16)
    cs = jnp.sum(a, axis=0, keepdims=True)
    bwd = _dot_t(a, x_blk_ref[...])

    @pl.when(i == 0)
    def _():
        cs_ref[...] = cs
        b1_ref[...] = bwd

    @pl.when(i != 0)
    def _():
        cs_ref[...] += cs
        b1_ref[...] += bwd


def _p2_body(a_ref, dcinv_row_ref, dcinv_col_ref, t1f16_full_ref, x_blk_ref,
             b1_blk_ref, t2f16_ref, t2b_ref):
    i = pl.program_id(0)
    a = a_ref[...]
    p116 = _p1_support(a)
    t2f = 2.0 * _dot(p116, t1f16_full_ref[...]) - x_blk_ref[...]
    t2f16_ref[...] = t2f.astype(bf16)
    q16 = (a * dcinv_row_ref[...]).astype(bf16)
    t1b16 = (dcinv_col_ref[...] * b1_blk_ref[...]).astype(bf16)
    bwd = _dot_t(q16, t1b16)

    @pl.when(i == 0)
    def _():
        t2b_ref[...] = bwd

    @pl.when(i != 0)
    def _():
        t2b_ref[...] += bwd


def _pw0_body(x_ref, t1f16_ref, t2f16_ref, b1_ref, dcinv_col_ref, t2b_ref,
              w5u_ref, w5c_ref, bgu_ref, bc_ref, h32_ref, h16_ref):
    x = x_ref[...]
    t0 = x.astype(bf16).astype(f32)
    t1 = t1f16_ref[...].astype(f32)
    t2 = t2f16_ref[...].astype(f32)
    t3 = (dcinv_col_ref[...] * b1_ref[...]).astype(bf16).astype(f32)
    t4 = (2.0 * t2b_ref[...] - x).astype(bf16).astype(f32)
    terms = (t0, t1, t2, t3, t4)
    for b in range(B):
        accu = jnp.broadcast_to(bgu_ref[...], (R, U))
        accc = jnp.broadcast_to(bc_ref[...], (R, U))
        for m, t in enumerate(terms):
            col = t[:, b:b + 1]
            accu = accu + col * w5u_ref[m:m + 1, :].astype(f32)
            accc = accc + col * w5c_ref[m:m + 1, :].astype(f32)
        u = jax.nn.sigmoid(accu)
        c = jnp.tanh(accc)
        h1 = (1.0 - u) * c
        h32_ref[:, b * U:(b + 1) * U] = h1
        h16_ref[:, b * U:(b + 1) * U] = h1.astype(bf16)


def _p3_body(a_ref, dcinv_row_ref, h16_full_ref, h16_blk_ref,
             t1f16_ref, s1b_ref):
    i = pl.program_id(0)
    a = a_ref[...]
    p116 = _p1_support(a)
    t1f16_ref[...] = _dot(p116, h16_full_ref[...]).astype(bf16)
    q16 = (a * dcinv_row_ref[...]).astype(bf16)
    bwd = _dot_t(q16, h16_blk_ref[...])

    @pl.when(i == 0)
    def _():
        s1b_ref[...] = bwd

    @pl.when(i != 0)
    def _():
        s1b_ref[...] += bwd


def _p4_body(a_ref, dcinv_row_ref, t1f16_full_ref, h32_blk_ref, s1b_blk_ref,
             t2f16_ref, s2b_ref):
    i = pl.program_id(0)
    a = a_ref[...]
    p116 = _p1_support(a)
    fwd = 2.0 * _dot(p116, t1f16_full_ref[...]) - h32_blk_ref[...]
    t2f16_ref[...] = fwd.astype(bf16)
    q16 = (a * dcinv_row_ref[...]).astype(bf16)
    bwd = _dot_t(q16, s1b_blk_ref[...].astype(bf16))

    @pl.when(i == 0)
    def _():
        s2b_ref[...] = bwd

    @pl.when(i != 0)
    def _():
        s2b_ref[...] += bwd


def _pw1_body(h32_ref, t1f16_ref, t2f16_ref, s1b_ref, s2b_ref,
              wgu_ref, wc_ref, bgu_ref, bc_ref, wp_ref, bp_ref,
              hid_ref, proj_ref):
    h = h32_ref[...]
    t2b = 2.0 * s2b_ref[...] - h
    h16 = h.astype(bf16)
    t1f16 = t1f16_ref[...]
    t2f16 = t2f16_ref[...]
    t1b16 = s1b_ref[...].astype(bf16)
    t2b16 = t2b.astype(bf16)
    wgu = wgu_ref[...]
    wc = wc_ref[...]
    wp = wp_ref[...].astype(f32)
    for b in range(B):
        sl = slice(b * U, (b + 1) * U)
        xb = jnp.concatenate(
            [h16[:, sl], t1f16[:, sl], t2f16[:, sl], t1b16[:, sl],
             t2b16[:, sl]], axis=1)
        u = jax.nn.sigmoid(_dot(xb, wgu) + bgu_ref[...])
        c = jnp.tanh(_dot(xb, wc) + bc_ref[...])
        h2b = (1.0 - u) * c
        hid_ref[0, b, :, :] = h[:, sl]
        hid_ref[1, b, :, :] = h2b
        proj_ref[:, b] = (jnp.sum(h2b.astype(bf16).astype(f32) * wp, axis=1)
                          + bp_ref[0, 0])


def _blk(shape):
    return pl.BlockSpec(shape, lambda i: (i,) + (0,) * (len(shape) - 1))


def _full(shape):
    return pl.BlockSpec(shape, lambda i: (0,) * len(shape))


@jax.jit
def _run(x, adj, Wg0, bg0, Wc0, bc0, Wg1, bg1, Wc1, bc1, Wp, bp):
    x16 = x.astype(bf16)

    idx0 = jnp.arange(5) * (1 + U)
    W5u = Wg0[idx0][:, U:].astype(bf16)
    W5c = Wc0[idx0].astype(bf16)
    idx1 = (jnp.arange(5)[:, None] * (2 * U) + jnp.arange(U)[None, :]).reshape(-1)
    Wg1u = Wg1[idx1][:, U:].astype(bf16)
    Wc1f = Wc1[idx1].astype(bf16)
    bg0u = bg0[U:].reshape(1, U)
    bc0r = bc0.reshape(1, U)
    bg1u = bg1[U:].reshape(1, U)
    bc1r = bc1.reshape(1, U)
    wp_t = Wp.reshape(1, U).astype(bf16)
    bp_r = bp.reshape(1, 1)

    grid = (NB,)

    t1f16, cs, b1 = pl.pallas_call(
        _p1_body,
        grid=grid,
        in_specs=[_blk((R, N)), _full((N, 4)), _blk((R, 4))],
        out_specs=[_blk((R, 4)), _full((1, N)), _full((N, 4))],
        out_shape=[jax.ShapeDtypeStruct((N, 4), bf16),
                   jax.ShapeDtypeStruct((1, N), f32),
                   jax.ShapeDtypeStruct((N, 4), f32)],
    )(adj, x16, x)

    dcinv_row = _inv(cs)
    dcinv_col = dcinv_row.reshape(N, 1)

    t2f16, t2b = pl.pallas_call(
        _p2_body,
        grid=grid,
        in_specs=[_blk((R, N)), _full((1, N)), _blk((R, 1)), _full((N, 4)),
                  _blk((R, 4)), _blk((R, 4))],
        out_specs=[_blk((R, 4)), _full((N, 4))],
        out_shape=[jax.ShapeDtypeStruct((N, 4), bf16),
                   jax.ShapeDtypeStruct((N, 4), f32)],
    )(adj, dcinv_row, dcinv_col, t1f16, x, b1)

    h32, h16 = pl.pallas_call(
        _pw0_body,
        grid=grid,
        in_specs=[_blk((R, 4)), _blk((R, 4)), _blk((R, 4)), _blk((R, 4)),
                  _blk((R, 1)), _blk((R, 4)), _full((5, U)), _full((5, U)),
                  _full((1, U)), _full((1, U))],
        out_specs=[_blk((R, BU)), _blk((R, BU))],
        out_shape=[jax.ShapeDtypeStruct((N, BU), f32),
                   jax.ShapeDtypeStruct((N, BU), bf16)],
    )(x, t1f16, t2f16, b1, dcinv_col, t2b, W5u, W5c, bg0u, bc0r)

    t1f16c1, s1b = pl.pallas_call(
        _p3_body,
        grid=grid,
        in_specs=[_blk((R, N)), _full((1, N)), _full((N, BU)), _blk((R, BU))],
        out_specs=[_blk((R, BU)), _full((N, BU))],
        out_shape=[jax.ShapeDtypeStruct((N, BU), bf16),
                   jax.ShapeDtypeStruct((N, BU), f32)],
    )(adj, dcinv_row, h16, h16)

    t2f16c1, s2b = pl.pallas_call(
        _p4_body,
        grid=grid,
        in_specs=[_blk((R, N)), _full((1, N)), _full((N, BU)), _blk((R, BU)),
                  _blk((R, BU))],
        out_specs=[_blk((R, BU)), _full((N, BU))],
        out_shape=[jax.ShapeDtypeStruct((N, BU), bf16),
                   jax.ShapeDtypeStruct((N, BU), f32)],
    )(adj, dcinv_row, t1f16c1, h32, s1b)

    hid4, proj = pl.pallas_call(
        _pw1_body,
        grid=grid,
        in_specs=[_blk((R, BU))] * 5 + [
                  _full((5 * U, U)), _full((5 * U, U)),
                  _full((1, U)), _full((1, U)), _full((1, U)),
                  _full((1, 1))],
        out_specs=[pl.BlockSpec((2, B, R, U), lambda i: (0, 0, i, 0)),
                   _blk((R, B))],
        out_shape=[jax.ShapeDtypeStruct((2, B, N, U), f32),
                   jax.ShapeDtypeStruct((N, B), f32)],
    )(h32, t1f16c1, t2f16c1, s1b, s2b, Wg1u, Wc1f, bg1u, bc1r, wp_t, bp_r)

    return proj.T, hid4.reshape(2, B, N * U)


def kernel(inputs, adj_mx, forward_index, Wg0, bg0, Wc0, bc0,
           Wg1, bg1, Wc1, bc1, Wp, bp):
    x = inputs.T.astype(f32)
    return _run(x, adj_mx, Wg0, bg0, Wc0, bc0, Wg1, bg1, Wc1, bc1, Wp, bp)


# speedup vs baseline: 2.0354x; 1.0401x over previous
import jax
import jax.numpy as jnp
from jax.experimental import pallas as pl

N = 3960
U = 64
B = 4
R = 440
NB = N // R
BU = B * U

f32 = jnp.float32
bf16 = jnp.bfloat16


def _dot(a, b):
    return jax.lax.dot_general(a, b, (((1,), (0,)), ((), ())),
                               preferred_element_type=f32)


def _dot_t(a, b):
    return jax.lax.dot_general(a, b, (((0,), (0,)), ((), ())),
                               preferred_element_type=f32)


def _inv(d):
    return jnp.where(d > 0, 1.0 / d, 0.0)


def _p1_support(a):
    rs = jnp.sum(a, axis=1, keepdims=True)
    return (a * _inv(rs)).astype(bf16)


def _p1_body(a_ref, x16_full_ref, x_blk_ref, t1f16_ref, cs_ref, b1_ref):
    i = pl.program_id(0)
    a = a_ref[...]
    p116 = _p1_support(a)
    t1f16_ref[...] = _dot(p116, x16_full_ref[...]).astype(bf16)
    cs = jnp.sum(a, axis=0, keepdims=True)
    bwd = _dot_t(a, x_blk_ref[...])

    @pl.when(i == 0)
    def _():
        cs_ref[...] = cs
        b1_ref[...] = bwd

    @pl.when(i != 0)
    def _():
        cs_ref[...] += cs
        b1_ref[...] += bwd


def _p2_body(a_ref, dcinv_row_ref, dcinv_col_ref, t1f16_full_ref, x_blk_ref,
             b1_blk_ref, t2f16_ref, t2b_ref):
    i = pl.program_id(0)
    a = a_ref[...]
    p116 = _p1_support(a)
    t2f = 2.0 * _dot(p116, t1f16_full_ref[...]) - x_blk_ref[...]
    t2f16_ref[...] = t2f.astype(bf16)
    q16 = (a * dcinv_row_ref[...]).astype(bf16)
    t1b16 = (dcinv_col_ref[...] * b1_blk_ref[...]).astype(bf16)
    bwd = _dot_t(q16, t1b16)

    @pl.when(i == 0)
    def _():
        t2b_ref[...] = bwd

    @pl.when(i != 0)
    def _():
        t2b_ref[...] += bwd


def _pw0_body(x_ref, t1f16_ref, t2f16_ref, b1_ref, dcinv_col_ref, t2b_ref,
              w5u_ref, w5c_ref, bgu_ref, bc_ref, h32_ref, h16_ref):
    x = x_ref[...]
    t0 = x.astype(bf16).astype(f32)
    t1 = t1f16_ref[...].astype(f32)
    t2 = t2f16_ref[...].astype(f32)
    t3 = (dcinv_col_ref[...] * b1_ref[...]).astype(bf16).astype(f32)
    t4 = (2.0 * t2b_ref[...] - x).astype(bf16).astype(f32)
    terms = (t0, t1, t2, t3, t4)
    for b in range(B):
        accu = jnp.broadcast_to(bgu_ref[...], (R, U))
        accc = jnp.broadcast_to(bc_ref[...], (R, U))
        for m, t in enumerate(terms):
            col = t[:, b:b + 1]
            accu = accu + col * w5u_ref[m:m + 1, :].astype(f32)
            accc = accc + col * w5c_ref[m:m + 1, :].astype(f32)
        u = jax.nn.sigmoid(accu)
        c = jnp.tanh(accc)
        h1 = (1.0 - u) * c
        h32_ref[:, b * U:(b + 1) * U] = h1
        h16_ref[:, b * U:(b + 1) * U] = h1.astype(bf16)


def _p3_body(a_ref, dcinv_row_ref, h16_full_ref, h16_blk_ref,
             t1f16_ref, s1b_ref):
    i = pl.program_id(0)
    a = a_ref[...]
    p116 = _p1_support(a)
    t1f16_ref[...] = _dot(p116, h16_full_ref[...]).astype(bf16)
    q16 = (a * dcinv_row_ref[...]).astype(bf16)
    bwd = _dot_t(q16, h16_blk_ref[...])

    @pl.when(i == 0)
    def _():
        s1b_ref[...] = bwd

    @pl.when(i != 0)
    def _():
        s1b_ref[...] += bwd


def _p4_body(a_ref, dcinv_row_ref, t1f16_full_ref, h32_blk_ref, s1b_blk_ref,
             t2f16_ref, s2b_ref):
    i = pl.program_id(0)
    a = a_ref[...]
    p116 = _p1_support(a)
    fwd = 2.0 * _dot(p116, t1f16_full_ref[...]) - h32_blk_ref[...]
    t2f16_ref[...] = fwd.astype(bf16)
    q16 = (a * dcinv_row_ref[...]).astype(bf16)
    bwd = _dot_t(q16, s1b_blk_ref[...].astype(bf16))

    @pl.when(i == 0)
    def _():
        s2b_ref[...] = bwd

    @pl.when(i != 0)
    def _():
        s2b_ref[...] += bwd


def _pw1_body(h32_ref, t1f16_ref, t2f16_ref, s1b_ref, s2b_ref,
              wgu_ref, wc_ref, bgu_ref, bc_ref, wp_ref, bp_ref,
              hid_ref, proj_ref):
    h = h32_ref[...]
    t2b = 2.0 * s2b_ref[...] - h
    h16 = h.astype(bf16)
    t1f16 = t1f16_ref[...]
    t2f16 = t2f16_ref[...]
    t1b16 = s1b_ref[...].astype(bf16)
    t2b16 = t2b.astype(bf16)
    wgu = wgu_ref[...]
    wc = wc_ref[...]
    wp = wp_ref[...].astype(f32)
    for b in range(B):
        sl = slice(b * U, (b + 1) * U)
        xb = jnp.concatenate(
            [h16[:, sl], t1f16[:, sl], t2f16[:, sl], t1b16[:, sl],
             t2b16[:, sl]], axis=1)
        u = jax.nn.sigmoid(_dot(xb, wgu) + bgu_ref[...])
        c = jnp.tanh(_dot(xb, wc) + bc_ref[...])
        h2b = (1.0 - u) * c
        hid_ref[0, b, :, :] = h[:, sl]
        hid_ref[1, b, :, :] = h2b
        proj_ref[:, b] = (jnp.sum(h2b.astype(bf16).astype(f32) * wp, axis=1)
                          + bp_ref[0, 0])


def _blk(shape):
    return pl.BlockSpec(shape, lambda i: (i,) + (0,) * (len(shape) - 1))


def _full(shape):
    return pl.BlockSpec(shape, lambda i: (0,) * len(shape))


@jax.jit
def _run(x, adj, Wg0, bg0, Wc0, bc0, Wg1, bg1, Wc1, bc1, Wp, bp):
    x16 = x.astype(bf16)

    idx0 = jnp.arange(5) * (1 + U)
    W5u = Wg0[idx0][:, U:].astype(bf16)
    W5c = Wc0[idx0].astype(bf16)
    idx1 = (jnp.arange(5)[:, None] * (2 * U) + jnp.arange(U)[None, :]).reshape(-1)
    Wg1u = Wg1[idx1][:, U:].astype(bf16)
    Wc1f = Wc1[idx1].astype(bf16)
    bg0u = bg0[U:].reshape(1, U)
    bc0r = bc0.reshape(1, U)
    bg1u = bg1[U:].reshape(1, U)
    bc1r = bc1.reshape(1, U)
    wp_t = Wp.reshape(1, U).astype(bf16)
    bp_r = bp.reshape(1, 1)

    grid = (NB,)

    t1f16, cs, b1 = pl.pallas_call(
        _p1_body,
        grid=grid,
        in_specs=[_blk((R, N)), _full((N, 4)), _blk((R, 4))],
        out_specs=[_blk((R, 4)), _full((1, N)), _full((N, 4))],
        out_shape=[jax.ShapeDtypeStruct((N, 4), bf16),
                   jax.ShapeDtypeStruct((1, N), f32),
                   jax.ShapeDtypeStruct((N, 4), f32)],
    )(adj, x16, x)

    dcinv_row = _inv(cs)
    dcinv_col = dcinv_row.reshape(N, 1)

    t2f16, t2b = pl.pallas_call(
        _p2_body,
        grid=grid,
        in_specs=[_blk((R, N)), _full((1, N)), _blk((R, 1)), _full((N, 4)),
                  _blk((R, 4)), _blk((R, 4))],
        out_specs=[_blk((R, 4)), _full((N, 4))],
        out_shape=[jax.ShapeDtypeStruct((N, 4), bf16),
                   jax.ShapeDtypeStruct((N, 4), f32)],
    )(adj, dcinv_row, dcinv_col, t1f16, x, b1)

    h32, h16 = pl.pallas_call(
        _pw0_body,
        grid=grid,
        in_specs=[_blk((R, 4)), _blk((R, 4)), _blk((R, 4)), _blk((R, 4)),
                  _blk((R, 1)), _blk((R, 4)), _full((5, U)), _full((5, U)),
                  _full((1, U)), _full((1, U))],
        out_specs=[_blk((R, BU)), _blk((R, BU))],
        out_shape=[jax.ShapeDtypeStruct((N, BU), f32),
                   jax.ShapeDtypeStruct((N, BU), bf16)],
    )(x, t1f16, t2f16, b1, dcinv_col, t2b, W5u, W5c, bg0u, bc0r)

    t1f16c1, s1b = pl.pallas_call(
        _p3_body,
        grid=grid,
        in_specs=[_blk((R, N)), _full((1, N)), _full((N, BU)), _blk((R, BU))],
        out_specs=[_blk((R, BU)), _full((N, BU))],
        out_shape=[jax.ShapeDtypeStruct((N, BU), bf16),
                   jax.ShapeDtypeStruct((N, BU), f32)],
    )(adj, dcinv_row, h16, h16)

    t2f16c1, s2b = pl.pallas_call(
        _p4_body,
        grid=grid,
        in_specs=[_blk((R, N)), _full((1, N)), _full((N, BU)), _blk((R, BU)),
                  _blk((R, BU))],
        out_specs=[_blk((R, BU)), _full((N, BU))],
        out_shape=[jax.ShapeDtypeStruct((N, BU), bf16),
                   jax.ShapeDtypeStruct((N, BU), f32)],
    )(adj, dcinv_row, t1f16c1, h32, s1b)

    hid4, proj = pl.pallas_call(
        _pw1_body,
        grid=grid,
        in_specs=[_blk((R, BU))] * 5 + [
                  _full((5 * U, U)), _full((5 * U, U)),
                  _full((1, U)), _full((1, U)), _full((1, U)),
                  _full((1, 1))],
        out_specs=[pl.BlockSpec((2, B, R, U), lambda i: (0, 0, i, 0)),
                   _blk((R, B))],
        out_shape=[jax.ShapeDtypeStruct((2, B, N, U), f32),
                   jax.ShapeDtypeStruct((N, B), f32)],
    )(h32, t1f16c1, t2f16c1, s1b, s2b, Wg1u, Wc1f, bg1u, bc1r, wp_t, bp_r)

    return proj.T, hid4.reshape(2, B, N * U)


def kernel(inputs, adj_mx, forward_index, Wg0, bg0, Wc0, bc0,
           Wg1, bg1, Wc1, bc1, Wp, bp):
    x = inputs.T.astype(f32)
    return _run(x, adj_mx, Wg0, bg0, Wc0, bc0, Wg1, bg1, Wc1, bc1, Wp, bp)


# speedup vs baseline: 2.0389x; 1.0018x over previous
import jax
import jax.numpy as jnp
from jax.experimental import pallas as pl

N = 3960
U = 64
B = 4
R = 792
NB = N // R
BU = B * U

f32 = jnp.float32
bf16 = jnp.bfloat16


def _dot(a, b):
    return jax.lax.dot_general(a, b, (((1,), (0,)), ((), ())),
                               preferred_element_type=f32)


def _dot_t(a, b):
    return jax.lax.dot_general(a, b, (((0,), (0,)), ((), ())),
                               preferred_element_type=f32)


def _inv(d):
    return jnp.where(d > 0, 1.0 / d, 0.0)


def _p1_support(a):
    rs = jnp.sum(a, axis=1, keepdims=True)
    return (a * _inv(rs)).astype(bf16)


def _p1_body(a_ref, x16_full_ref, x_blk_ref, t1f16_ref, cs_ref, b1_ref):
    i = pl.program_id(0)
    a = a_ref[...]
    p116 = _p1_support(a)
    t1f16_ref[...] = _dot(p116, x16_full_ref[...]).astype(bf16)
    cs = jnp.sum(a, axis=0, keepdims=True)
    bwd = _dot_t(a, x_blk_ref[...])

    @pl.when(i == 0)
    def _():
        cs_ref[...] = cs
        b1_ref[...] = bwd

    @pl.when(i != 0)
    def _():
        cs_ref[...] += cs
        b1_ref[...] += bwd


def _p2_body(a_ref, dcinv_row_ref, dcinv_col_ref, t1f16_full_ref, x_blk_ref,
             b1_blk_ref, t2f16_ref, t2b_ref):
    i = pl.program_id(0)
    a = a_ref[...]
    p116 = _p1_support(a)
    t2f = 2.0 * _dot(p116, t1f16_full_ref[...]) - x_blk_ref[...]
    t2f16_ref[...] = t2f.astype(bf16)
    q16 = (a * dcinv_row_ref[...]).astype(bf16)
    t1b16 = (dcinv_col_ref[...] * b1_blk_ref[...]).astype(bf16)
    bwd = _dot_t(q16, t1b16)

    @pl.when(i == 0)
    def _():
        t2b_ref[...] = bwd

    @pl.when(i != 0)
    def _():
        t2b_ref[...] += bwd


def _pw0_body(x_ref, t1f16_ref, t2f16_ref, b1_ref, dcinv_col_ref, t2b_ref,
              w5u_ref, w5c_ref, bgu_ref, bc_ref, h32_ref, h16_ref):
    x = x_ref[...]
    t0 = x.astype(bf16).astype(f32)
    t1 = t1f16_ref[...].astype(f32)
    t2 = t2f16_ref[...].astype(f32)
    t3 = (dcinv_col_ref[...] * b1_ref[...]).astype(bf16).astype(f32)
    t4 = (2.0 * t2b_ref[...] - x).astype(bf16).astype(f32)
    terms = (t0, t1, t2, t3, t4)
    for b in range(B):
        accu = jnp.broadcast_to(bgu_ref[...], (R, U))
        accc = jnp.broadcast_to(bc_ref[...], (R, U))
        for m, t in enumerate(terms):
            col = t[:, b:b + 1]
            accu = accu + col * w5u_ref[m:m + 1, :].astype(f32)
            accc = accc + col * w5c_ref[m:m + 1, :].astype(f32)
        u = jax.nn.sigmoid(accu)
        c = jnp.tanh(accc)
        h1 = (1.0 - u) * c
        h32_ref[:, b * U:(b + 1) * U] = h1
        h16_ref[:, b * U:(b + 1) * U] = h1.astype(bf16)


def _p3_body(a_ref, dcinv_row_ref, h16_full_ref, h16_blk_ref,
             t1f16_ref, s1b_ref):
    i = pl.program_id(0)
    a = a_ref[...]
    p116 = _p1_support(a)
    t1f16_ref[...] = _dot(p116, h16_full_ref[...]).astype(bf16)
    q16 = (a * dcinv_row_ref[...]).astype(bf16)
    bwd = _dot_t(q16, h16_blk_ref[...])

    @pl.when(i == 0)
    def _():
        s1b_ref[...] = bwd

    @pl.when(i != 0)
    def _():
        s1b_ref[...] += bwd


def _p4_body(a_ref, dcinv_row_ref, t1f16_full_ref, h32_blk_ref, s1b_blk_ref,
             t2f16_ref, s2b_ref):
    i = pl.program_id(0)
    a = a_ref[...]
    p116 = _p1_support(a)
    fwd = 2.0 * _dot(p116, t1f16_full_ref[...]) - h32_blk_ref[...]
    t2f16_ref[...] = fwd.astype(bf16)
    q16 = (a * dcinv_row_ref[...]).astype(bf16)
    bwd = _dot_t(q16, s1b_blk_ref[...].astype(bf16))

    @pl.when(i == 0)
    def _():
        s2b_ref[...] = bwd

    @pl.when(i != 0)
    def _():
        s2b_ref[...] += bwd


def _pw1_body(h32_ref, t1f16_ref, t2f16_ref, s1b_ref, s2b_ref,
              wgu_ref, wc_ref, bgu_ref, bc_ref, wp_ref, bp_ref,
              hid_ref, proj_ref):
    h = h32_ref[...]
    t2b = 2.0 * s2b_ref[...] - h
    h16 = h.astype(bf16)
    t1f16 = t1f16_ref[...]
    t2f16 = t2f16_ref[...]
    t1b16 = s1b_ref[...].astype(bf16)
    t2b16 = t2b.astype(bf16)
    wgu = wgu_ref[...]
    wc = wc_ref[...]
    wp = wp_ref[...].astype(f32)
    for b in range(B):
        sl = slice(b * U, (b + 1) * U)
        xb = jnp.concatenate(
            [h16[:, sl], t1f16[:, sl], t2f16[:, sl], t1b16[:, sl],
             t2b16[:, sl]], axis=1)
        u = jax.nn.sigmoid(_dot(xb, wgu) + bgu_ref[...])
        c = jnp.tanh(_dot(xb, wc) + bc_ref[...])
        h2b = (1.0 - u) * c
        hid_ref[0, b, :, :] = h[:, sl]
        hid_ref[1, b, :, :] = h2b
        proj_ref[:, b] = (jnp.sum(h2b.astype(bf16).astype(f32) * wp, axis=1)
                          + bp_ref[0, 0])


def _blk(shape):
    return pl.BlockSpec(shape, lambda i: (i,) + (0,) * (len(shape) - 1))


def _full(shape):
    return pl.BlockSpec(shape, lambda i: (0,) * len(shape))


@jax.jit
def _run(x, adj, Wg0, bg0, Wc0, bc0, Wg1, bg1, Wc1, bc1, Wp, bp):
    x16 = x.astype(bf16)

    idx0 = jnp.arange(5) * (1 + U)
    W5u = Wg0[idx0][:, U:].astype(bf16)
    W5c = Wc0[idx0].astype(bf16)
    idx1 = (jnp.arange(5)[:, None] * (2 * U) + jnp.arange(U)[None, :]).reshape(-1)
    Wg1u = Wg1[idx1][:, U:].astype(bf16)
    Wc1f = Wc1[idx1].astype(bf16)
    bg0u = bg0[U:].reshape(1, U)
    bc0r = bc0.reshape(1, U)
    bg1u = bg1[U:].reshape(1, U)
    bc1r = bc1.reshape(1, U)
    wp_t = Wp.reshape(1, U).astype(bf16)
    bp_r = bp.reshape(1, 1)

    grid = (NB,)

    t1f16, cs, b1 = pl.pallas_call(
        _p1_body,
        grid=grid,
        in_specs=[_blk((R, N)), _full((N, 4)), _blk((R, 4))],
        out_specs=[_blk((R, 4)), _full((1, N)), _full((N, 4))],
        out_shape=[jax.ShapeDtypeStruct((N, 4), bf16),
                   jax.ShapeDtypeStruct((1, N), f32),
                   jax.ShapeDtypeStruct((N, 4), f32)],
    )(adj, x16, x)

    dcinv_row = _inv(cs)
    dcinv_col = dcinv_row.reshape(N, 1)

    t2f16, t2b = pl.pallas_call(
        _p2_body,
        grid=grid,
        in_specs=[_blk((R, N)), _full((1, N)), _blk((R, 1)), _full((N, 4)),
                  _blk((R, 4)), _blk((R, 4))],
        out_specs=[_blk((R, 4)), _full((N, 4))],
        out_shape=[jax.ShapeDtypeStruct((N, 4), bf16),
                   jax.ShapeDtypeStruct((N, 4), f32)],
    )(adj, dcinv_row, dcinv_col, t1f16, x, b1)

    h32, h16 = pl.pallas_call(
        _pw0_body,
        grid=grid,
        in_specs=[_blk((R, 4)), _blk((R, 4)), _blk((R, 4)), _blk((R, 4)),
                  _blk((R, 1)), _blk((R, 4)), _full((5, U)), _full((5, U)),
                  _full((1, U)), _full((1, U))],
        out_specs=[_blk((R, BU)), _blk((R, BU))],
        out_shape=[jax.ShapeDtypeStruct((N, BU), f32),
                   jax.ShapeDtypeStruct((N, BU), bf16)],
    )(x, t1f16, t2f16, b1, dcinv_col, t2b, W5u, W5c, bg0u, bc0r)

    t1f16c1, s1b = pl.pallas_call(
        _p3_body,
        grid=grid,
        in_specs=[_blk((R, N)), _full((1, N)), _full((N, BU)), _blk((R, BU))],
        out_specs=[_blk((R, BU)), _full((N, BU))],
        out_shape=[jax.ShapeDtypeStruct((N, BU), bf16),
                   jax.ShapeDtypeStruct((N, BU), f32)],
    )(adj, dcinv_row, h16, h16)

    t2f16c1, s2b = pl.pallas_call(
        _p4_body,
        grid=grid,
        in_specs=[_blk((R, N)), _full((1, N)), _full((N, BU)), _blk((R, BU)),
                  _blk((R, BU))],
        out_specs=[_blk((R, BU)), _full((N, BU))],
        out_shape=[jax.ShapeDtypeStruct((N, BU), bf16),
                   jax.ShapeDtypeStruct((N, BU), f32)],
    )(adj, dcinv_row, t1f16c1, h32, s1b)

    hid4, proj = pl.pallas_call(
        _pw1_body,
        grid=grid,
        in_specs=[_blk((R, BU))] * 5 + [
                  _full((5 * U, U)), _full((5 * U, U)),
                  _full((1, U)), _full((1, U)), _full((1, U)),
                  _full((1, 1))],
        out_specs=[pl.BlockSpec((2, B, R, U), lambda i: (0, 0, i, 0)),
                   _blk((R, B))],
        out_shape=[jax.ShapeDtypeStruct((2, B, N, U), f32),
                   jax.ShapeDtypeStruct((N, B), f32)],
    )(h32, t1f16c1, t2f16c1, s1b, s2b, Wg1u, Wc1f, bg1u, bc1r, wp_t, bp_r)

    return proj.T, hid4.reshape(2, B, N * U)


def kernel(inputs, adj_mx, forward_index, Wg0, bg0, Wc0, bc0,
           Wg1, bg1, Wc1, bc1, Wp, bp):
    x = inputs.T.astype(f32)
    return _run(x, adj_mx, Wg0, bg0, Wc0, bc0, Wg1, bg1, Wc1, bc1, Wp, bp)
